# baseline (speedup 1.0000x reference)
import jax
import jax.numpy as jnp
from jax import lax
from jax.experimental import pallas as pl
from jax.experimental.pallas import tpu as pltpu
from jax.experimental.pallas import tpu_sc as plsc

N = 10000
NPAD = 10240
E = 320000
EPAD = 327680
CHUNK = 128
EROWS = EPAD // CHUNK
F_IN = 128
F3 = 96
FP = 128
P = 12
NPAIR = 6
NC = 2
NS = 16
L = 16

def _sc_mesh():
    return plsc.VectorSubcoreMesh(core_axis_name="c", subcore_axis_name="s",
                                  num_cores=NC, num_subcores=NS)


_DEG_ROWS_PER_TILE = EROWS // (NC * NS)
_STRIPE = NPAD // NS


def _deg_body(dst2d, out, dstv, ones, zb, hist):
    cid = lax.axis_index("c")
    sid = lax.axis_index("s")
    wid = sid * NC + cid

    def _fill_ones(r, carry):
        ones[r] = jnp.full((L,), 1.0, jnp.float32)
        return carry

    lax.fori_loop(0, CHUNK, _fill_ones, 0)

    def _fill_zb(r, carry):
        zb[r] = jnp.zeros((L,), jnp.float32)
        return carry

    lax.fori_loop(0, 80, _fill_zb, 0)

    pltpu.sync_copy(dst2d.at[pl.ds(wid * _DEG_ROWS_PER_TILE, _DEG_ROWS_PER_TILE)], dstv)
    for kk in range(_STRIPE // 80):
        pltpu.sync_copy(zb, hist.at[pl.ds(sid * _STRIPE + kk * 80, 80)])
    plsc.subcore_barrier()

    def _scatter(j, carry):
        pltpu.sync_copy(ones, hist.at[dstv.at[j]], add=True)
        return carry

    lax.fori_loop(0, _DEG_ROWS_PER_TILE, _scatter, 0)
    plsc.subcore_barrier()
    pltpu.sync_copy(hist.at[pl.ds(sid * _STRIPE, _STRIPE)],
                    out.at[cid, pl.ds(sid * _STRIPE, _STRIPE)])


def _deg_kernel(dst2d):
    return pl.kernel(
        _deg_body,
        out_type=jax.ShapeDtypeStruct((NC, NPAD, L), jnp.float32),
        mesh=_sc_mesh(),
        scratch_types=[
            pltpu.VMEM((_DEG_ROWS_PER_TILE, CHUNK), jnp.int32),
            pltpu.VMEM((CHUNK, L), jnp.float32),
            pltpu.VMEM((80, L), jnp.float32),
            pltpu.VMEM_SHARED((NPAD, L), jnp.float32),
        ],
        compiler_params=pltpu.CompilerParams(use_tc_tiling_on_sc=False),
    )(dst2d)


_AGG_ROWS_PER_TILE = EROWS // NS
_PPC = NPAIR // NC
_SN = 4
_RN = 2
_PF = _SN - _RN
_D = _RN - 1


def _agg_body(edges2d, qflat, out, ebuf, soff, rows, zb, agg, *sems):
    cid = lax.axis_index("c")
    sid = lax.axis_index("s")
    ei = sems[:_SN]
    gsem = sems[_SN:_SN + _RN]
    ssem = sems[_SN + _RN:]
    C = _AGG_ROWS_PER_TILE

    def _fill_zb(r, carry):
        for h in range(2):
            for t in range(4):
                zb[r, h, pl.ds(t * 32, 32)] = jnp.zeros((32,), jnp.bfloat16)
        return carry

    lax.fori_loop(0, 80, _fill_zb, 0)

    def _idx_load(j, b):
        row = sid * C + j
        return pltpu.async_copy(edges2d.at[pl.ds(row, 1)],
                                ebuf.at[pl.ds(b, 1)], ei[b])

    def _idx_wait(j, b):
        row = sid * C + j
        pltpu.make_async_copy(edges2d.at[pl.ds(row, 1)],
                              ebuf.at[pl.ds(b, 1)], ei[b]).wait()

    def _gather(u, rb, off):
        for t in range(CHUNK // L):
            soff[u, pl.ds(t * L, L)] = ebuf[u, 0, pl.ds(t * L, L)] + off
        pltpu.async_copy(qflat.at[soff.at[u]], rows.at[rb], gsem[rb])

    def _gather_wait(u, rb):
        pltpu.make_async_copy(qflat.at[soff.at[u]], rows.at[rb],
                              gsem[rb]).wait()

    def _scatter(u, rb):
        pltpu.async_copy(rows.at[rb], agg.at[ebuf.at[u, 1]], ssem[rb],
                         add=True)

    def _scatter_wait(u, rb):
        pltpu.make_async_copy(rows.at[rb], agg.at[ebuf.at[u, 1]],
                              ssem[rb]).wait()

    def _pos(j, u, k0, klast):
        rb = u % _RN
        if not (k0 and j < _RN):
            _scatter_wait((u - _RN) % _SN, rb)
        if not (klast and j + _PF >= C):
            _idx_load(j + _PF, (u + _PF) % _SN)
        _idx_wait(j, u % _SN)
        yield rb
        if not (k0 and j < _D):
            _gather_wait((u - _D) % _SN, (u - _D) % _RN)
            _scatter((u - _D) % _SN, (u - _D) % _RN)

    def _period(pi, carry):
        p = cid * _PPC + pi
        off = p * NPAD

        for kk in range(_STRIPE // 80):
            pltpu.sync_copy(zb, agg.at[pl.ds(sid * _STRIPE + kk * 80, 80)])
        for b in range(_PF):
            _idx_load(b, b)
        plsc.subcore_barrier()

        def _block(k, c2, k0=False, klast=False):
            for u in range(_SN):
                j = k * _SN + u
                step = _pos(j, u, k0, klast)
                rb = next(step)
                _gather(u, rb, off)
                for _ in step:
                    pass
            return c2

        _block(0, 0, k0=True)
        lax.fori_loop(1, C // _SN - 1, _block, 0)
        _block(C // _SN - 1, 0, klast=True)
        for d in range(_D):
            j2 = C - _D + d
            _gather_wait(j2 % _SN, j2 % _RN)
            _scatter(j2 % _SN, j2 % _RN)
        for r in range(_RN):
            j2 = C - _RN + r
            _scatter_wait(j2 % _SN, j2 % _RN)
        plsc.subcore_barrier()
        pltpu.sync_copy(agg.at[pl.ds(sid * _STRIPE, _STRIPE)],
                        out.at[p, pl.ds(sid * _STRIPE, _STRIPE)])
        return carry

    lax.fori_loop(0, _PPC, _period, 0)


def _agg_kernel(edges2d, qflat):
    return pl.kernel(
        _agg_body,
        out_type=jax.ShapeDtypeStruct((NPAIR, NPAD, 2, 128), jnp.bfloat16),
        mesh=_sc_mesh(),
        scratch_types=[
            pltpu.VMEM((_SN, 2, CHUNK), jnp.int32),
            pltpu.VMEM((_SN, CHUNK), jnp.int32),
            pltpu.VMEM((_RN, CHUNK, 2, 128), jnp.bfloat16),
            pltpu.VMEM((80, 2, 128), jnp.bfloat16),
            pltpu.VMEM_SHARED((NPAD, 2, 128), jnp.bfloat16),
        ] + [pltpu.SemaphoreType.DMA] * (_SN + 2 * _RN),
        compiler_params=pltpu.CompilerParams(use_tc_tiling_on_sc=False),
    )(edges2d, qflat)


_BLK = 1280
_NB = NPAD // _BLK
_BLKG = 1280
_NBG = NPAD // _BLKG


def _proj_body(deg_ref, xt_ref, wcat_ref, q_ref):
    deg = deg_ref[0, :, 0] + deg_ref[1, :, 0] + 1.0
    dinv = lax.rsqrt(deg)[:, None]
    z32 = jnp.zeros((_BLK, 128 - F3), jnp.float32)
    q0 = dinv * jnp.dot(xt_ref[0], wcat_ref[...],
                        preferred_element_type=jnp.float32)
    q1 = dinv * jnp.dot(xt_ref[1], wcat_ref[...],
                        preferred_element_type=jnp.float32)
    q_ref[0] = jnp.concatenate([q0, z32, q1, z32], axis=1).astype(jnp.bfloat16)


def _proj_call(deg_parts, xt, wcat):
    return pl.pallas_call(
        _proj_body,
        grid=(NPAIR, _NB),
        in_specs=[
            pl.BlockSpec((NC, _BLK, L), lambda p, i: (0, i, 0)),
            pl.BlockSpec((2, _BLK, F_IN), lambda p, i: (p, i, 0)),
            pl.BlockSpec((F_IN, F3), lambda p, i: (0, 0)),
        ],
        out_specs=pl.BlockSpec((1, _BLK, 256), lambda p, i: (p, i, 0)),
        out_shape=jax.ShapeDtypeStruct((NPAIR, NPAD, 256), jnp.bfloat16),
    )(deg_parts, xt, wcat)


def _gru_body(raw_ref, q_ref, deg_ref, att_ref, bcat_ref,
              lza_ref, lzb2_ref, lzbias_ref,
              lra_ref, lrb2_ref, lrbias_ref,
              lha_ref, lhb2_ref, lhbias_ref,
              wlin_ref, blin_ref, out_ref):
    att = att_ref[...]
    m = jnp.max(att)
    e = jnp.exp(att - m)
    probs = e / jnp.sum(e)

    dinv = lax.rsqrt(deg_ref[0, :, 0] + deg_ref[1, :, 0] + 1.0)[:, None]
    bcat = bcat_ref[...]
    lza, lzb2, lzbias = lza_ref[...], lzb2_ref[...], lzbias_ref[...]
    lra, lrb2, lrbias = lra_ref[...], lrb2_ref[...], lrbias_ref[...]
    lha, lhb2, lhbias = lha_ref[...], lhb2_ref[...], lhbias_ref[...]

    h = jnp.zeros((_BLKG, 32), jnp.float32)
    hacc = jnp.zeros((_BLKG, 32), jnp.float32)
    for p in range(P):
        pair, half = p // 2, p % 2
        rawp = raw_ref[pair][:, half * 128:half * 128 + F3].astype(jnp.float32)
        qp = q_ref[pair][:, half * 128:half * 128 + F3].astype(jnp.float32)
        g = dinv * (rawp + qp) + bcat
        gz = g[:, 0:32]
        gr = g[:, 32:64]
        gh = g[:, 64:96]
        z = jax.nn.sigmoid(
            jnp.dot(gz, lza, preferred_element_type=jnp.float32)
            + jnp.dot(h, lzb2, preferred_element_type=jnp.float32) + lzbias)
        r = jax.nn.sigmoid(
            jnp.dot(gr, lra, preferred_element_type=jnp.float32)
            + jnp.dot(h, lrb2, preferred_element_type=jnp.float32) + lrbias)
        ht = jnp.tanh(
            jnp.dot(gh, lha, preferred_element_type=jnp.float32)
            + jnp.dot(h * r, lhb2, preferred_element_type=jnp.float32) + lhbias)
        h = z * h + (1.0 - z) * ht
        hacc = hacc + probs[p:p + 1] * h
    hrelu = jnp.maximum(hacc, 0.0)
    out_ref[...] = (jnp.dot(hrelu, wlin_ref[...],
                            preferred_element_type=jnp.float32) + blin_ref[...])


def _gru_call(raw, q, deg_parts, att, bcat, lza, lzb2, lzbias, lra, lrb2, lrbias,
              lha, lhb2, lhbias, wlint, blin):
    full = lambda shape: pl.BlockSpec(shape, lambda i: tuple(0 for _ in shape))
    return pl.pallas_call(
        _gru_body,
        grid=(_NBG,),
        in_specs=[
            pl.BlockSpec((NPAIR, _BLKG, 256), lambda i: (0, i, 0)),
            pl.BlockSpec((NPAIR, _BLKG, 256), lambda i: (0, i, 0)),
            pl.BlockSpec((NC, _BLKG, L), lambda i: (0, i, 0)),
            full((P,)),
            full((F3,)),
            full((32, 32)), full((32, 32)), full((32,)),
            full((32, 32)), full((32, 32)), full((32,)),
            full((32, 32)), full((32, 32)), full((32,)),
            full((32, 2)), full((2,)),
        ],
        out_specs=pl.BlockSpec((_BLKG, 2), lambda i: (i, 0)),
        out_shape=jax.ShapeDtypeStruct((NPAD, 2), jnp.float32),
    )(raw, q, deg_parts, att, bcat, lza, lzb2, lzbias, lra, lrb2, lrbias,
      lha, lhb2, lhbias, wlint, blin)


def kernel(x, edge_index, task, attention, Wz, bz, Wr, br, Wh, bh,
           Lz, Lz_b, Lr, Lr_b, Lh, Lh_b, Wlin, blin):
    del task
    pad = jnp.full((EPAD - E,), N, jnp.int32)
    src2d = jnp.concatenate([edge_index[0], pad]).reshape(EROWS, CHUNK)
    dst2d = jnp.concatenate([edge_index[1], pad]).reshape(EROWS, CHUNK)
    edges2d = jnp.pad(jnp.stack([src2d, dst2d], axis=1),
                      ((0, _PF), (0, 0), (0, 0)), constant_values=N)
    xt = jnp.pad(jnp.transpose(x, (2, 0, 1)), ((0, 0), (0, NPAD - N), (0, 0)))
    wcat = jnp.concatenate([Wz, Wr, Wh], axis=1)
    bcat = jnp.concatenate([bz, br, bh])

    deg_parts = _deg_kernel(dst2d)
    q = _proj_call(deg_parts, xt, wcat)
    raw = _agg_kernel(edges2d, q.reshape(NPAIR * NPAD, 2, 128))
    out = _gru_call(raw.reshape(NPAIR, NPAD, 256), q, deg_parts, attention, bcat,
                    Lz[:, :32].T, Lz[:, 32:].T, Lz_b,
                    Lr[:, :32].T, Lr[:, 32:].T, Lr_b,
                    Lh[:, :32].T, Lh[:, 32:].T, Lh_b,
                    Wlin.T, blin)
    return out[:N]


# speedup vs baseline: 1.0163x; 1.0163x over previous
import jax
import jax.numpy as jnp
from jax import lax
from jax.experimental import pallas as pl
from jax.experimental.pallas import tpu as pltpu
from jax.experimental.pallas import tpu_sc as plsc

N = 10000
NPAD = 10240
E = 320000
EPAD = 327680
CHUNK = 128
EROWS = EPAD // CHUNK
F_IN = 128
F3 = 96
FP = 128
P = 12
NPAIR = 6
NC = 2
NS = 16
L = 16

def _sc_mesh():
    return plsc.VectorSubcoreMesh(core_axis_name="c", subcore_axis_name="s",
                                  num_cores=NC, num_subcores=NS)


_DEG_ROWS_PER_TILE = EROWS // (NC * NS)
_STRIPE = NPAD // NS


def _deg_body(dst2d, out, dstv, ones, zb, hist):
    cid = lax.axis_index("c")
    sid = lax.axis_index("s")
    wid = sid * NC + cid

    def _fill_ones(r, carry):
        for t in range(FP // L):
            ones[r, pl.ds(t * L, L)] = jnp.full((L,), 1.0, jnp.float32)
        return carry

    lax.fori_loop(0, CHUNK, _fill_ones, 0)

    def _fill_zb(r, carry):
        for t in range(FP // L):
            zb[r, pl.ds(t * L, L)] = jnp.zeros((L,), jnp.float32)
        return carry

    lax.fori_loop(0, 80, _fill_zb, 0)

    pltpu.sync_copy(dst2d.at[pl.ds(wid * _DEG_ROWS_PER_TILE, _DEG_ROWS_PER_TILE)], dstv)
    for kk in range(_STRIPE // 80):
        pltpu.sync_copy(zb, hist.at[pl.ds(sid * _STRIPE + kk * 80, 80)])
    plsc.subcore_barrier()

    def _scatter(j, carry):
        pltpu.sync_copy(ones, hist.at[dstv.at[j]], add=True)
        return carry

    lax.fori_loop(0, _DEG_ROWS_PER_TILE, _scatter, 0)
    plsc.subcore_barrier()
    pltpu.sync_copy(hist.at[pl.ds(sid * _STRIPE, _STRIPE)],
                    out.at[cid, pl.ds(sid * _STRIPE, _STRIPE)])


def _deg_kernel(dst2d):
    return pl.kernel(
        _deg_body,
        out_type=jax.ShapeDtypeStruct((NC, NPAD, FP), jnp.float32),
        mesh=_sc_mesh(),
        scratch_types=[
            pltpu.VMEM((_DEG_ROWS_PER_TILE, CHUNK), jnp.int32),
            pltpu.VMEM((CHUNK, FP), jnp.float32),
            pltpu.VMEM((80, FP), jnp.float32),
            pltpu.VMEM_SHARED((NPAD, FP), jnp.float32),
        ],
    )(dst2d)


_AGG_ROWS_PER_TILE = EROWS // NS
_PPC = NPAIR // NC
_SN = 4
_RN = 2
_PF = _SN - _RN
_D = _RN - 1


def _agg_body(edges2d, qflat, out, ebuf, soff, rows, zb, agg, *sems):
    cid = lax.axis_index("c")
    sid = lax.axis_index("s")
    ei = sems[:_SN]
    gsem = sems[_SN:_SN + _RN]
    ssem = sems[_SN + _RN:]
    C = _AGG_ROWS_PER_TILE

    def _fill_zb(r, carry):
        for h in range(2):
            for t in range(4):
                zb[r, h, pl.ds(t * 32, 32)] = jnp.zeros((32,), jnp.bfloat16)
        return carry

    lax.fori_loop(0, 80, _fill_zb, 0)

    def _idx_load(j, b):
        row = sid * C + j
        return pltpu.async_copy(edges2d.at[pl.ds(row, 1)],
                                ebuf.at[pl.ds(b, 1)], ei[b])

    def _idx_wait(j, b):
        row = sid * C + j
        pltpu.make_async_copy(edges2d.at[pl.ds(row, 1)],
                              ebuf.at[pl.ds(b, 1)], ei[b]).wait()

    def _gather(u, rb, off):
        for t in range(CHUNK // L):
            soff[u, pl.ds(t * L, L)] = ebuf[u, 0, pl.ds(t * L, L)] + off
        pltpu.async_copy(qflat.at[soff.at[u]], rows.at[rb], gsem[rb])

    def _gather_wait(u, rb):
        pltpu.make_async_copy(qflat.at[soff.at[u]], rows.at[rb],
                              gsem[rb]).wait()

    def _scatter(u, rb):
        pltpu.async_copy(rows.at[rb], agg.at[ebuf.at[u, 1]], ssem[rb],
                         add=True)

    def _scatter_wait(u, rb):
        pltpu.make_async_copy(rows.at[rb], agg.at[ebuf.at[u, 1]],
                              ssem[rb]).wait()

    def _pos(j, u, k0, klast):
        rb = u % _RN
        if not (k0 and j < _RN):
            _scatter_wait((u - _RN) % _SN, rb)
        if not (klast and j + _PF >= C):
            _idx_load(j + _PF, (u + _PF) % _SN)
        _idx_wait(j, u % _SN)
        yield rb
        if not (k0 and j < _D):
            _gather_wait((u - _D) % _SN, (u - _D) % _RN)
            _scatter((u - _D) % _SN, (u - _D) % _RN)

    def _period(pi, carry):
        p = cid * _PPC + pi
        off = p * NPAD

        for kk in range(_STRIPE // 80):
            pltpu.sync_copy(zb, agg.at[pl.ds(sid * _STRIPE + kk * 80, 80)])
        for b in range(_PF):
            _idx_load(b, b)
        plsc.subcore_barrier()

        def _block(k, c2, k0=False, klast=False):
            for u in range(_SN):
                j = k * _SN + u
                step = _pos(j, u, k0, klast)
                rb = next(step)
                _gather(u, rb, off)
                for _ in step:
                    pass
            return c2

        _block(0, 0, k0=True)
        lax.fori_loop(1, C // _SN - 1, _block, 0)
        _block(C // _SN - 1, 0, klast=True)
        for d in range(_D):
            j2 = C - _D + d
            _gather_wait(j2 % _SN, j2 % _RN)
            _scatter(j2 % _SN, j2 % _RN)
        for r in range(_RN):
            j2 = C - _RN + r
            _scatter_wait(j2 % _SN, j2 % _RN)
        plsc.subcore_barrier()
        pltpu.sync_copy(agg.at[pl.ds(sid * _STRIPE, _STRIPE)],
                        out.at[p, pl.ds(sid * _STRIPE, _STRIPE)])
        return carry

    lax.fori_loop(0, _PPC, _period, 0)


def _agg_kernel(edges2d, qflat):
    return pl.kernel(
        _agg_body,
        out_type=jax.ShapeDtypeStruct((NPAIR, NPAD, 2, 128), jnp.bfloat16),
        mesh=_sc_mesh(),
        scratch_types=[
            pltpu.VMEM((_SN, 2, CHUNK), jnp.int32),
            pltpu.VMEM((_SN, CHUNK), jnp.int32),
            pltpu.VMEM((_RN, CHUNK, 2, 128), jnp.bfloat16),
            pltpu.VMEM((80, 2, 128), jnp.bfloat16),
            pltpu.VMEM_SHARED((NPAD, 2, 128), jnp.bfloat16),
        ] + [pltpu.SemaphoreType.DMA] * (_SN + 2 * _RN),
        compiler_params=pltpu.CompilerParams(use_tc_tiling_on_sc=False),
    )(edges2d, qflat)


_BLK = 1280
_NB = NPAD // _BLK
_BLKG = 640
_NBG = NPAD // _BLKG


def _proj_body(deg_ref, xt_ref, wcat_ref, q_ref):
    deg = deg_ref[0, :, 0] + deg_ref[1, :, 0] + 1.0
    dinv = lax.rsqrt(deg)[:, None]
    z32 = jnp.zeros((_BLK, 128 - F3), jnp.float32)
    q0 = dinv * jnp.dot(xt_ref[0], wcat_ref[...],
                        preferred_element_type=jnp.float32)
    q1 = dinv * jnp.dot(xt_ref[1], wcat_ref[...],
                        preferred_element_type=jnp.float32)
    q_ref[0] = jnp.concatenate([q0, z32, q1, z32], axis=1).astype(jnp.bfloat16)


def _proj_call(deg_parts, xt, wcat):
    return pl.pallas_call(
        _proj_body,
        grid=(NPAIR, _NB),
        in_specs=[
            pl.BlockSpec((NC, _BLK, FP), lambda p, i: (0, i, 0)),
            pl.BlockSpec((2, _BLK, F_IN), lambda p, i: (p, i, 0)),
            pl.BlockSpec((F_IN, F3), lambda p, i: (0, 0)),
        ],
        out_specs=pl.BlockSpec((1, _BLK, 256), lambda p, i: (p, i, 0)),
        out_shape=jax.ShapeDtypeStruct((NPAIR, NPAD, 256), jnp.bfloat16),
    )(deg_parts, xt, wcat)


def _gru_body(raw_ref, q_ref, deg_ref, att_ref, bcat_ref,
              lza_ref, lzb2_ref, lzbias_ref,
              lra_ref, lrb2_ref, lrbias_ref,
              lha_ref, lhb2_ref, lhbias_ref,
              wlin_ref, blin_ref, out_ref):
    att = att_ref[...]
    m = jnp.max(att)
    e = jnp.exp(att - m)
    probs = e / jnp.sum(e)

    dinv = lax.rsqrt(deg_ref[0, :, 0] + deg_ref[1, :, 0] + 1.0)[:, None]
    bcat = bcat_ref[...]
    lza, lzb2, lzbias = lza_ref[...], lzb2_ref[...], lzbias_ref[...]
    lra, lrb2, lrbias = lra_ref[...], lrb2_ref[...], lrbias_ref[...]
    lha, lhb2, lhbias = lha_ref[...], lhb2_ref[...], lhbias_ref[...]

    h = jnp.zeros((_BLKG, 32), jnp.float32)
    hacc = jnp.zeros((_BLKG, 32), jnp.float32)
    for p in range(P):
        pair, half = p // 2, p % 2
        rawp = raw_ref[pair][:, half * 128:half * 128 + F3].astype(jnp.float32)
        qp = q_ref[pair][:, half * 128:half * 128 + F3].astype(jnp.float32)
        g = dinv * (rawp + qp) + bcat
        gz = g[:, 0:32]
        gr = g[:, 32:64]
        gh = g[:, 64:96]
        z = jax.nn.sigmoid(
            jnp.dot(gz, lza, preferred_element_type=jnp.float32)
            + jnp.dot(h, lzb2, preferred_element_type=jnp.float32) + lzbias)
        r = jax.nn.sigmoid(
            jnp.dot(gr, lra, preferred_element_type=jnp.float32)
            + jnp.dot(h, lrb2, preferred_element_type=jnp.float32) + lrbias)
        ht = jnp.tanh(
            jnp.dot(gh, lha, preferred_element_type=jnp.float32)
            + jnp.dot(h * r, lhb2, preferred_element_type=jnp.float32) + lhbias)
        h = z * h + (1.0 - z) * ht
        hacc = hacc + probs[p:p + 1] * h
    hrelu = jnp.maximum(hacc, 0.0)
    out_ref[...] = (jnp.dot(hrelu, wlin_ref[...],
                            preferred_element_type=jnp.float32) + blin_ref[...])


def _gru_call(raw, q, deg_parts, att, bcat, lza, lzb2, lzbias, lra, lrb2, lrbias,
              lha, lhb2, lhbias, wlint, blin):
    full = lambda shape: pl.BlockSpec(shape, lambda i: tuple(0 for _ in shape))
    return pl.pallas_call(
        _gru_body,
        grid=(_NBG,),
        in_specs=[
            pl.BlockSpec((NPAIR, _BLKG, 256), lambda i: (0, i, 0)),
            pl.BlockSpec((NPAIR, _BLKG, 256), lambda i: (0, i, 0)),
            pl.BlockSpec((NC, _BLKG, FP), lambda i: (0, i, 0)),
            full((P,)),
            full((F3,)),
            full((32, 32)), full((32, 32)), full((32,)),
            full((32, 32)), full((32, 32)), full((32,)),
            full((32, 32)), full((32, 32)), full((32,)),
            full((32, 2)), full((2,)),
        ],
        out_specs=pl.BlockSpec((_BLKG, 2), lambda i: (i, 0)),
        out_shape=jax.ShapeDtypeStruct((NPAD, 2), jnp.float32),
    )(raw, q, deg_parts, att, bcat, lza, lzb2, lzbias, lra, lrb2, lrbias,
      lha, lhb2, lhbias, wlint, blin)


def kernel(x, edge_index, task, attention, Wz, bz, Wr, br, Wh, bh,
           Lz, Lz_b, Lr, Lr_b, Lh, Lh_b, Wlin, blin):
    del task
    pad = jnp.full((EPAD - E,), N, jnp.int32)
    src2d = jnp.concatenate([edge_index[0], pad]).reshape(EROWS, CHUNK)
    dst2d = jnp.concatenate([edge_index[1], pad]).reshape(EROWS, CHUNK)
    edges2d = jnp.pad(jnp.stack([src2d, dst2d], axis=1),
                      ((0, _PF), (0, 0), (0, 0)), constant_values=N)
    xt = jnp.pad(jnp.transpose(x, (2, 0, 1)), ((0, 0), (0, NPAD - N), (0, 0)))
    wcat = jnp.concatenate([Wz, Wr, Wh], axis=1)
    bcat = jnp.concatenate([bz, br, bh])

    deg_parts = _deg_kernel(dst2d)
    q = _proj_call(deg_parts, xt, wcat)
    raw = _agg_kernel(edges2d, q.reshape(NPAIR * NPAD, 2, 128))
    out = _gru_call(raw.reshape(NPAIR, NPAD, 256), q, deg_parts, attention, bcat,
                    Lz[:, :32].T, Lz[:, 32:].T, Lz_b,
                    Lr[:, :32].T, Lr[:, 32:].T, Lr_b,
                    Lh[:, :32].T, Lh[:, 32:].T, Lh_b,
                    Wlin.T, blin)
    return out[:N]
